# Initial kernel scaffold; baseline (speedup 1.0000x reference)
import jax
import jax.numpy as jnp
from jax.experimental import pallas as pl


def kernel(x):
    raise NotImplementedError("write your pallas kernel here")


# speedup vs baseline: 9.6180x; 9.6180x over previous
import functools

import jax
import jax.numpy as jnp
import numpy as np
from jax.experimental import pallas as pl
from jax.experimental.pallas import tpu as pltpu

_ALPHA = 0.99
_INIT_HI = -60.0
_INIT_LO = -90.0
_VAR0 = 40.0 ** 2

_B_BLK = 16
_SUB = 256


def _time_block(t: int) -> int:
    for nc in (4, 2, 1):
        if t % nc == 0:
            tb = t // nc
            if nc == 1 or tb % 8 == 0:
                return tb
    return t


def _consts():
    j = np.arange(_SUB, dtype=np.float64)
    l01 = np.where(j[:, None] >= j[None, :], 1.0 - _ALPHA, 0.0)
    p1 = np.broadcast_to((_ALPHA ** -(j + 1))[:, None], (_SUB, 128))
    g = np.broadcast_to((_ALPHA ** ((j + 1) / 2))[:, None], (_SUB, 128))
    return (jnp.asarray(l01, dtype=jnp.float32),
            jnp.asarray(np.ascontiguousarray(p1), dtype=jnp.float32),
            jnp.asarray(np.ascontiguousarray(g), dtype=jnp.float32))


def _body(tb, f, bb, x_ref, l_ref, p1_ref, g_ref, o_ref, mu_sc, var_sc):
    lanes = bb * f
    t = pl.program_id(1)

    @pl.when(t == 0)
    def _init():
        lane = jax.lax.broadcasted_iota(jnp.int32, (1, lanes), 1)
        step = (_INIT_LO - _INIT_HI) / (f - 1)
        mu_sc[...] = _INIT_HI + (lane % f).astype(jnp.float32) * step
        var_sc[...] = jnp.full((1, lanes), _VAR0, dtype=jnp.float32)

    reps = max(1, lanes // 128)
    l01 = l_ref[...]
    p1_full = jnp.tile(p1_ref[...], (1, reps))[:, :lanes]
    g_full = jnp.tile(g_ref[...], (1, reps))[:, :lanes]

    cmu = mu_sc[...]
    cvar = var_sc[...]
    for st in range(0, tb, _SUB):
        ln = min(_SUB, tb - st)
        xc = jnp.concatenate(
            [x_ref[i, 0, st:st + ln, :] for i in range(bb)], axis=-1)
        if ln == _SUB:
            lsub, p1, g = l01, p1_full, g_full
        else:
            lsub, p1, g = l01[:ln, :ln], p1_full[:ln], g_full[:ln]
        xs = xc * p1
        ms = cmu + jnp.dot(lsub, xs, preferred_element_type=jnp.float32)
        h = (xs - ms) * g
        vars_ = cvar + jnp.dot(lsub, h * h,
                               preferred_element_type=jnp.float32)
        out = h * jax.lax.rsqrt(vars_)
        for i in range(bb):
            o_ref[i, 0, st:st + ln, :] = out[:, i * f:(i + 1) * f]
        end = _ALPHA ** ln
        cmu = ms[ln - 1:ln, :] * end
        cvar = vars_[ln - 1:ln, :] * end

    mu_sc[...] = cmu
    var_sc[...] = cvar


def kernel(x):
    b, t_total, f = x.shape
    tb = _time_block(t_total)
    nc = t_total // tb
    bb = _B_BLK if b % _B_BLK == 0 else 1
    lanes = bb * f
    l01, p1, g = _consts()

    x4 = x.reshape(b, nc, tb, f)
    body = functools.partial(_body, tb, f, bb)
    out = pl.pallas_call(
        body,
        grid=(b // bb, nc),
        in_specs=[
            pl.BlockSpec((bb, 1, tb, f), lambda i, j: (i, j, 0, 0)),
            pl.BlockSpec(l01.shape, lambda i, j: (0, 0)),
            pl.BlockSpec(p1.shape, lambda i, j: (0, 0)),
            pl.BlockSpec(g.shape, lambda i, j: (0, 0)),
        ],
        out_specs=pl.BlockSpec((bb, 1, tb, f), lambda i, j: (i, j, 0, 0)),
        out_shape=jax.ShapeDtypeStruct((b, nc, tb, f), jnp.float32),
        scratch_shapes=[
            pltpu.VMEM((1, lanes), jnp.float32),
            pltpu.VMEM((1, lanes), jnp.float32),
        ],
        compiler_params=pltpu.CompilerParams(
            dimension_semantics=("parallel", "arbitrary"),
        ),
    )(x4, l01, p1, g)
    return out.reshape(b, t_total, f)


# speedup vs baseline: 9.6477x; 1.0031x over previous
import functools

import jax
import jax.numpy as jnp
import numpy as np
from jax.experimental import pallas as pl
from jax.experimental.pallas import tpu as pltpu

_ALPHA = 0.99
_INIT_HI = -60.0
_INIT_LO = -90.0
_VAR0 = 40.0 ** 2

_B_BLK = 16
_SUB = 256
_LGRP = 8


def _time_block(t: int, bb: int, f: int) -> int:
    best = t
    for nc in range(1, t + 1):
        if t % nc:
            continue
        best = t // nc
        if bb * best * f * 4 <= 4 * 1024 * 1024:
            break
    return best


def _consts():
    j = np.arange(_SUB, dtype=np.float64)
    l01 = np.where(j[:, None] >= j[None, :], 1.0 - _ALPHA, 0.0)
    p1 = np.broadcast_to((_ALPHA ** -(j + 1))[:, None], (_SUB, 128))
    g = np.broadcast_to((_ALPHA ** ((j + 1) / 2))[:, None], (_SUB, 128))
    return (jnp.asarray(l01, dtype=jnp.float32),
            jnp.asarray(np.ascontiguousarray(p1), dtype=jnp.float32),
            jnp.asarray(np.ascontiguousarray(g), dtype=jnp.float32))


def _body(tb, f, bb, x_ref, l_ref, p1_ref, g_ref, o_ref, mu_sc, var_sc):
    lanes = bb * f
    t = pl.program_id(1)

    @pl.when(t == 0)
    def _init():
        lane = jax.lax.broadcasted_iota(jnp.int32, (1, lanes), 1)
        step = (_INIT_LO - _INIT_HI) / (f - 1)
        mu_sc[...] = _INIT_HI + (lane % f).astype(jnp.float32) * step
        var_sc[...] = jnp.full((1, lanes), _VAR0, dtype=jnp.float32)

    reps = max(1, lanes // 128)
    l01 = l_ref[...]
    p1_full = jnp.tile(p1_ref[...], (1, reps))[:, :lanes]
    g_full = jnp.tile(g_ref[...], (1, reps))[:, :lanes]

    grp = min(_LGRP, bb)
    glanes = grp * f
    cmu = [mu_sc[:, gs * f:gs * f + glanes] for gs in range(0, bb, grp)]
    cvar = [var_sc[:, gs * f:gs * f + glanes] for gs in range(0, bb, grp)]
    for st in range(0, tb, _SUB):
        ln = min(_SUB, tb - st)
        if ln == _SUB:
            lsub, p1, g = l01, p1_full, g_full
        else:
            lsub, p1, g = l01[:ln, :ln], p1_full[:ln], g_full[:ln]
        end = _ALPHA ** ln
        for gi, gs in enumerate(range(0, bb, grp)):
            xc = jnp.concatenate(
                [x_ref[i, 0, st:st + ln, :] for i in range(gs, gs + grp)],
                axis=-1)
            xs = xc * p1[:, :glanes]
            ms = cmu[gi] + jnp.dot(lsub, xs,
                                   preferred_element_type=jnp.float32)
            h = (xs - ms) * g[:, :glanes]
            vars_ = cvar[gi] + jnp.dot(lsub, h * h,
                                       preferred_element_type=jnp.float32)
            out = h * jax.lax.rsqrt(vars_)
            for i in range(gs, gs + grp):
                o_ref[i, 0, st:st + ln, :] = out[:, (i - gs) * f:
                                                 (i - gs + 1) * f]
            cmu[gi] = ms[ln - 1:ln, :] * end
            cvar[gi] = vars_[ln - 1:ln, :] * end

    for gi, gs in enumerate(range(0, bb, grp)):
        mu_sc[:, gs * f:gs * f + glanes] = cmu[gi]
        var_sc[:, gs * f:gs * f + glanes] = cvar[gi]


def kernel(x):
    b, t_total, f = x.shape
    bb = _B_BLK if b % _B_BLK == 0 else 1
    tb = _time_block(t_total, bb, f)
    nc = t_total // tb
    lanes = bb * f
    l01, p1, g = _consts()

    x4 = x.reshape(b, nc, tb, f)
    body = functools.partial(_body, tb, f, bb)
    out = pl.pallas_call(
        body,
        grid=(b // bb, nc),
        in_specs=[
            pl.BlockSpec((bb, 1, tb, f), lambda i, j: (i, j, 0, 0)),
            pl.BlockSpec(l01.shape, lambda i, j: (0, 0)),
            pl.BlockSpec(p1.shape, lambda i, j: (0, 0)),
            pl.BlockSpec(g.shape, lambda i, j: (0, 0)),
        ],
        out_specs=pl.BlockSpec((bb, 1, tb, f), lambda i, j: (i, j, 0, 0)),
        out_shape=jax.ShapeDtypeStruct((b, nc, tb, f), jnp.float32),
        scratch_shapes=[
            pltpu.VMEM((1, lanes), jnp.float32),
            pltpu.VMEM((1, lanes), jnp.float32),
        ],
        compiler_params=pltpu.CompilerParams(
            dimension_semantics=("parallel", "arbitrary"),
        ),
    )(x4, l01, p1, g)
    return out.reshape(b, t_total, f)


# speedup vs baseline: 41.1391x; 4.2641x over previous
import functools

import jax
import jax.numpy as jnp
from jax.experimental import pallas as pl
from jax.experimental.pallas import tpu as pltpu

_ALPHA = 0.99
_C = 1.0 - _ALPHA
_INIT_HI = -60.0
_INIT_LO = -90.0
_VAR0 = 40.0 ** 2


def _t_chunk(t: int, f: int, bl: int) -> int:
    budget = 4 * 1024 * 1024 // (f * bl * 4)
    best = 1
    for s in range(1, t + 1):
        if t % s == 0 and s <= budget:
            best = s
    return best


def _body(st, f, bl, x_ref, o_ref, mu_sc, v2_sc):
    j = pl.program_id(1)

    @pl.when(j == 0)
    def _init():
        frow = jax.lax.broadcasted_iota(jnp.int32, (f, bl), 0)
        step = (_INIT_LO - _INIT_HI) / (f - 1)
        mu_sc[...] = _INIT_HI + frow.astype(jnp.float32) * step
        v2_sc[...] = jnp.full((f, bl), _VAR0 / (_ALPHA * _ALPHA),
                              dtype=jnp.float32)

    unroll = 5 if st % 5 == 0 else (4 if st % 4 == 0 else 1)

    def step_fn(k, carry):
        mu, v2 = carry
        base = k * unroll
        for u in range(unroll):
            xv = x_ref[base + u]
            e = xv - mu
            mu = mu + _C * e
            v2 = _ALPHA * v2 + _C * (e * e)
            o_ref[base + u] = e * jax.lax.rsqrt(v2)
        return mu, v2

    mu, v2 = jax.lax.fori_loop(0, st // unroll, step_fn,
                               (mu_sc[...], v2_sc[...]))
    mu_sc[...] = mu
    v2_sc[...] = v2


def kernel(x):
    b, t_total, f = x.shape
    xt = jnp.transpose(x, (1, 2, 0))
    bl = 128 if b % 128 == 0 else b
    nb = b // bl
    st = _t_chunk(t_total, f, bl)
    nt = t_total // st

    body = functools.partial(_body, st, f, bl)
    out_t = pl.pallas_call(
        body,
        grid=(nb, nt),
        in_specs=[pl.BlockSpec((st, f, bl), lambda i, j: (j, 0, i))],
        out_specs=pl.BlockSpec((st, f, bl), lambda i, j: (j, 0, i)),
        out_shape=jax.ShapeDtypeStruct((t_total, f, b), jnp.float32),
        scratch_shapes=[
            pltpu.VMEM((f, bl), jnp.float32),
            pltpu.VMEM((f, bl), jnp.float32),
        ],
        compiler_params=pltpu.CompilerParams(
            dimension_semantics=("parallel", "arbitrary"),
        ),
    )(xt)
    return jnp.transpose(out_t, (2, 0, 1))


# speedup vs baseline: 41.9953x; 1.0208x over previous
import functools

import jax
import jax.numpy as jnp
from jax.experimental import pallas as pl
from jax.experimental.pallas import tpu as pltpu

_ALPHA = 0.99
_C = 1.0 - _ALPHA
_INIT_HI = -60.0
_INIT_LO = -90.0
_VAR0 = 40.0 ** 2


def _t_chunk(t: int, f: int, bl: int) -> int:
    budget = min(128, 4 * 1024 * 1024 // (f * bl * 4))
    best = 1
    for s in range(1, t + 1):
        if t % s == 0 and s <= budget:
            best = s
    return best


def _body(st, f, bl, x_ref, o_ref, mu_sc, v2_sc):
    j = pl.program_id(1)

    @pl.when(j == 0)
    def _init():
        frow = jax.lax.broadcasted_iota(jnp.int32, (f, bl), 0)
        step = (_INIT_LO - _INIT_HI) / (f - 1)
        mu_sc[...] = _INIT_HI + frow.astype(jnp.float32) * step
        v2_sc[...] = jnp.full((f, bl), _VAR0 / (_ALPHA * _ALPHA),
                              dtype=jnp.float32)

    mu = mu_sc[...]
    v2 = v2_sc[...]
    for idx in range(st):
        xv = x_ref[idx]
        e = xv - mu
        mu = mu + _C * e
        v2 = _ALPHA * v2 + _C * (e * e)
        o_ref[idx] = e * jax.lax.rsqrt(v2)
    mu_sc[...] = mu
    v2_sc[...] = v2


def kernel(x):
    b, t_total, f = x.shape
    xt = jnp.transpose(x, (1, 2, 0))
    bl = 128 if b % 128 == 0 else b
    nb = b // bl
    st = _t_chunk(t_total, f, bl)
    nt = t_total // st

    body = functools.partial(_body, st, f, bl)
    out_t = pl.pallas_call(
        body,
        grid=(nb, nt),
        in_specs=[pl.BlockSpec((st, f, bl), lambda i, j: (j, 0, i))],
        out_specs=pl.BlockSpec((st, f, bl), lambda i, j: (j, 0, i)),
        out_shape=jax.ShapeDtypeStruct((t_total, f, b), jnp.float32),
        scratch_shapes=[
            pltpu.VMEM((f, bl), jnp.float32),
            pltpu.VMEM((f, bl), jnp.float32),
        ],
        compiler_params=pltpu.CompilerParams(
            dimension_semantics=("parallel", "arbitrary"),
        ),
    )(xt)
    return jnp.transpose(out_t, (2, 0, 1))
